# Initial kernel scaffold; baseline (speedup 1.0000x reference)
import jax
import jax.numpy as jnp
from jax.experimental import pallas as pl


def kernel(t):
    raise NotImplementedError("write your pallas kernel here")


# speedup vs baseline: 3.1504x; 3.1504x over previous
import functools

import jax
import jax.numpy as jnp
from jax import lax
from jax.experimental import pallas as pl
from jax.experimental.pallas import tpu as pltpu
from jax.experimental.pallas import tpu_sc as plsc

_DIM = 128
_NUM_POS = 100000

_ROW_BLOCK = 2000


def _table_body(out_ref):
    i = pl.program_id(0)
    r = jax.lax.broadcasted_iota(jnp.int32, (_ROW_BLOCK, _DIM), 0)
    r = (r + i * _ROW_BLOCK).astype(jnp.float32)
    b = r / 10000.0
    c = jax.lax.broadcasted_iota(jnp.int32, (_ROW_BLOCK, _DIM), 1)
    k = c // 2
    e = k.astype(jnp.float32) / _DIM
    phase = jnp.where(k == 0, 1.0, jnp.exp(e * jnp.log(b)))
    out_ref[...] = jnp.where(c % 2 == 0, jnp.sin(phase), jnp.cos(phase))


def _build_table():
    return pl.pallas_call(
        _table_body,
        out_shape=jax.ShapeDtypeStruct((_NUM_POS, _DIM), jnp.float32),
        grid=(_NUM_POS // _ROW_BLOCK,),
        out_specs=pl.BlockSpec((_ROW_BLOCK, _DIM), lambda i: (i, 0)),
    )()


_B = 16384 * 50
_L = 128
_K = 5
_NW = 32
_B_PER_W = _B // _NW
_NSTREAM = _B_PER_W // _L
_NGROUP = _NSTREAM // _K
_G_ROWS = _K * _L


def _gather_kernel(table_hbm, idx_hbm, out_hbm, idx_v, rows_v, gsem):
    nc = 2
    wid = lax.axis_index("s") * nc + lax.axis_index("c")
    base = wid * _B_PER_W
    pltpu.sync_copy(idx_hbm.at[pl.ds(wid * _NSTREAM, _NSTREAM)], idx_v)

    def group(g, carry):
        copies = []
        for j in range(_K):
            s = g * _K + j
            copies.append(
                pltpu.async_copy(
                    table_hbm.at[idx_v.at[s]],
                    rows_v.at[pl.ds(j * _L, _L)],
                    gsem,
                )
            )
        for cp in copies:
            cp.wait()
        pltpu.sync_copy(
            rows_v, out_hbm.at[pl.ds(base + g * _G_ROWS, _G_ROWS)]
        )
        return carry

    lax.fori_loop(0, _NGROUP, group, 0)


def _gather(table, idx_flat):
    mesh = plsc.VectorSubcoreMesh(core_axis_name="c", subcore_axis_name="s")
    f = functools.partial(
        pl.kernel,
        mesh=mesh,
        out_type=jax.ShapeDtypeStruct((_B, _DIM), jnp.float32),
        scratch_types=[
            pltpu.VMEM((_NSTREAM, _L), jnp.int32),
            pltpu.VMEM((_G_ROWS, _DIM), jnp.float32),
            pltpu.SemaphoreType.DMA,
        ],
    )(_gather_kernel)
    return f(table, idx_flat)


def kernel(t):
    table = _build_table()
    idx = t.reshape(_B // _L, _L).astype(jnp.int32)
    out = _gather(table, idx)
    return out.reshape(t.shape[0], t.shape[1], _DIM)


# speedup vs baseline: 4.8983x; 1.5548x over previous
import functools

import jax
import jax.numpy as jnp
from jax import lax
from jax.experimental import pallas as pl
from jax.experimental.pallas import tpu as pltpu
from jax.experimental.pallas import tpu_sc as plsc

_DIM = 128
_NUM_POS = 100000

_ROW_BLOCK = 2000


def _table_body(out_ref):
    i = pl.program_id(0)
    r = jax.lax.broadcasted_iota(jnp.int32, (_ROW_BLOCK, _DIM), 0)
    r = (r + i * _ROW_BLOCK).astype(jnp.float32)
    b = r / 10000.0
    c = jax.lax.broadcasted_iota(jnp.int32, (_ROW_BLOCK, _DIM), 1)
    k = c // 2
    e = k.astype(jnp.float32) / _DIM
    phase = jnp.where(k == 0, 1.0, jnp.exp(e * jnp.log(b)))
    out_ref[...] = jnp.where(c % 2 == 0, jnp.sin(phase), jnp.cos(phase))


def _build_table():
    return pl.pallas_call(
        _table_body,
        out_shape=jax.ShapeDtypeStruct((_NUM_POS, _DIM), jnp.float32),
        grid=(_NUM_POS // _ROW_BLOCK,),
        out_specs=pl.BlockSpec((_ROW_BLOCK, _DIM), lambda i: (i, 0)),
    )()


_T_ROWS = 16384
_T_COLS = 50
_B = _T_ROWS * _T_COLS
_NW = 32
_TR_PER_W = _T_ROWS // _NW
_B_PER_W = _TR_PER_W * _T_COLS
_G = 8
_NGROUP = _TR_PER_W // _G


_PAD = 56


def _gather_kernel(table_hbm, idx_hbm, out_hbm, idx_v, rows_v, gsem):
    nc = 2
    wid = lax.axis_index("s") * nc + lax.axis_index("c")
    pltpu.sync_copy(
        idx_hbm.at[pl.ds(wid * _TR_PER_W * _PAD, _TR_PER_W * _PAD)], idx_v
    )

    def group(g, carry):
        copies = []
        for j in range(_G):
            s = g * _G + j
            copies.append(
                pltpu.async_copy(
                    table_hbm.at[idx_v.at[pl.ds(s * _PAD, _T_COLS)]],
                    rows_v.at[j],
                    gsem,
                )
            )
        for cp in copies:
            cp.wait()
        pltpu.sync_copy(
            rows_v, out_hbm.at[pl.ds(wid * _TR_PER_W + g * _G, _G)]
        )
        return carry

    lax.fori_loop(0, _NGROUP, group, 0)


def _gather(table, idx_flat):
    mesh = plsc.VectorSubcoreMesh(core_axis_name="c", subcore_axis_name="s")
    f = functools.partial(
        pl.kernel,
        mesh=mesh,
        out_type=jax.ShapeDtypeStruct((_T_ROWS, _T_COLS, _DIM), jnp.float32),
        scratch_types=[
            pltpu.VMEM((_TR_PER_W * _PAD,), jnp.int32),
            pltpu.VMEM((_G, _T_COLS, _DIM), jnp.float32),
            pltpu.SemaphoreType.DMA,
        ],
        compiler_params=pltpu.CompilerParams(use_tc_tiling_on_sc=True),
    )(_gather_kernel)
    return f(table, idx_flat)


def kernel(t):
    table = _build_table()
    idx = jnp.pad(t.astype(jnp.int32), ((0, 0), (0, _PAD - _T_COLS)))
    return _gather(table, idx.reshape(-1))


# speedup vs baseline: 7.6295x; 1.5576x over previous
import functools

import jax
import jax.numpy as jnp
from jax import lax
from jax.experimental import pallas as pl
from jax.experimental.pallas import tpu as pltpu
from jax.experimental.pallas import tpu_sc as plsc

_DIM = 128
_NUM_POS = 100000

_ROW_BLOCK = 2000
_HALF_PI = 1.5707963267948966


def _table_body(out_ref):
    i = pl.program_id(0)
    r = jax.lax.broadcasted_iota(jnp.int32, (_ROW_BLOCK, _DIM), 0)
    r = (r + i * _ROW_BLOCK).astype(jnp.float32)
    b = r / 10000.0
    c = jax.lax.broadcasted_iota(jnp.int32, (_ROW_BLOCK, _DIM), 1)
    k = c // 2
    e = k.astype(jnp.float32) / _DIM
    phase = jnp.where(k == 0, 1.0, jnp.exp(e * jnp.log(b)))
    phase = phase + jnp.where(c % 2 == 0, 0.0, _HALF_PI)
    out_ref[...] = jnp.sin(phase)


def _build_table():
    return pl.pallas_call(
        _table_body,
        out_shape=jax.ShapeDtypeStruct((_NUM_POS, _DIM), jnp.float32),
        grid=(_NUM_POS // _ROW_BLOCK,),
        out_specs=pl.BlockSpec((_ROW_BLOCK, _DIM), lambda i: (i, 0)),
    )()


_T_ROWS = 16384
_T_COLS = 50
_B = _T_ROWS * _T_COLS
_NW = 32
_I_PER_W = _T_ROWS // _NW
_NSTREAM = _I_PER_W // _DIM


def _gather_kernel(table_hbm, idx_hbm, out_hbm, idx_v, rows_v, gsem):
    nc = 2
    wid = lax.axis_index("s") * nc + lax.axis_index("c")
    base_i = wid * _I_PER_W

    def column(j, carry):
        pltpu.sync_copy(
            idx_hbm.at[pl.ds(j * _T_ROWS + base_i, _I_PER_W)], idx_v
        )
        copies = []
        for k in range(_NSTREAM):
            copies.append(
                pltpu.async_copy(
                    table_hbm.at[idx_v.at[pl.ds(k * _DIM, _DIM)]],
                    rows_v.at[pl.ds(k * _DIM, _DIM)],
                    gsem,
                )
            )
        for cp in copies:
            cp.wait()
        pltpu.sync_copy(
            rows_v, out_hbm.at[pl.ds(j * _T_ROWS + base_i, _I_PER_W)]
        )
        return carry

    lax.fori_loop(0, _T_COLS, column, 0)


def _gather(table, idx_flat):
    mesh = plsc.VectorSubcoreMesh(core_axis_name="c", subcore_axis_name="s")
    f = functools.partial(
        pl.kernel,
        mesh=mesh,
        out_type=jax.ShapeDtypeStruct((_B, _DIM), jnp.float32),
        scratch_types=[
            pltpu.VMEM((_I_PER_W,), jnp.int32),
            pltpu.VMEM((_I_PER_W, _DIM), jnp.float32),
            pltpu.SemaphoreType.DMA,
        ],
    )(_gather_kernel)
    return f(table, idx_flat)


def kernel(t):
    table = _build_table()
    idx = t.T.astype(jnp.int32).reshape(-1)
    out = _gather(table, idx)
    return out.reshape(_T_COLS, _T_ROWS, _DIM).transpose(1, 0, 2)


# speedup vs baseline: 10.6197x; 1.3919x over previous
import functools

import jax
import jax.numpy as jnp
from jax import lax
from jax.experimental import pallas as pl
from jax.experimental.pallas import tpu as pltpu
from jax.experimental.pallas import tpu_sc as plsc

_DIM = 128
_NUM_POS = 100000

_ROW_BLOCK = 2000
_PI = 3.141592653589793
_HALF_PI = _PI / 2


def _fast_sin(x):
    sign = jnp.where(x > _PI, -1.0, 1.0)
    y = jnp.where(x > _PI, x - _PI, x)
    y = jnp.where(y > _HALF_PI, _PI - y, y)
    s = y * y
    p = jnp.float32(1.0 / 362880)
    p = p * s + jnp.float32(-1.0 / 5040)
    p = p * s + jnp.float32(1.0 / 120)
    p = p * s + jnp.float32(-1.0 / 6)
    p = p * s + 1.0
    return sign * y * p


def _table_body(out_ref):
    i = pl.program_id(0)
    r = jax.lax.broadcasted_iota(jnp.int32, (_ROW_BLOCK, _DIM), 0)
    r = (r + i * _ROW_BLOCK).astype(jnp.float32)
    b = r * jnp.float32(1.0 / 10000.0)
    c = jax.lax.broadcasted_iota(jnp.int32, (_ROW_BLOCK, _DIM), 1)
    k = c // 2
    e = k.astype(jnp.float32) * jnp.float32(1.0 / _DIM)
    phase = jnp.where(k == 0, 1.0, jnp.exp2(e * jnp.log2(b)))
    phase = phase + jnp.where(c % 2 == 0, 0.0, _HALF_PI)
    out_ref[...] = _fast_sin(phase)


def _build_table():
    return pl.pallas_call(
        _table_body,
        out_shape=jax.ShapeDtypeStruct((_NUM_POS, _DIM), jnp.float32),
        grid=(_NUM_POS // _ROW_BLOCK,),
        out_specs=pl.BlockSpec((_ROW_BLOCK, _DIM), lambda i: (i, 0)),
    )()


_T_ROWS = 16384
_T_COLS = 50
_B = _T_ROWS * _T_COLS
_NW = 32
_I_PER_W = _T_ROWS // _NW
_H = _I_PER_W // 2


def _gather_kernel(
    table_hbm, idx_hbm, out_hbm, idx_v, rows_a, rows_b, gsem_a, gsem_b,
    osem_a, osem_b
):
    nc = 2
    wid = lax.axis_index("s") * nc + lax.axis_index("c")
    base_i = wid * _I_PER_W

    def half(j, rows_v, gsem, osem, off):
        @pl.when(j > 0)
        def _():
            pltpu.make_async_copy(
                rows_v,
                out_hbm.at[pl.ds((j - 1) * _T_ROWS + base_i + off, _H)],
                osem,
            ).wait()

        c0 = pltpu.async_copy(
            table_hbm.at[idx_v.at[pl.ds(off, _DIM)]],
            rows_v.at[pl.ds(0, _DIM)],
            gsem,
        )
        c1 = pltpu.async_copy(
            table_hbm.at[idx_v.at[pl.ds(off + _DIM, _DIM)]],
            rows_v.at[pl.ds(_DIM, _DIM)],
            gsem,
        )
        c0.wait()
        c1.wait()
        pltpu.async_copy(
            rows_v, out_hbm.at[pl.ds(j * _T_ROWS + base_i + off, _H)], osem
        )

    def column(j, carry):
        pltpu.sync_copy(
            idx_hbm.at[pl.ds(j * _T_ROWS + base_i, _I_PER_W)], idx_v
        )
        half(j, rows_a, gsem_a, osem_a, 0)
        half(j, rows_b, gsem_b, osem_b, _H)
        return carry

    lax.fori_loop(0, _T_COLS, column, 0)
    j_last = _T_COLS - 1
    pltpu.make_async_copy(
        rows_a, out_hbm.at[pl.ds(j_last * _T_ROWS + base_i, _H)], osem_a
    ).wait()
    pltpu.make_async_copy(
        rows_b, out_hbm.at[pl.ds(j_last * _T_ROWS + base_i + _H, _H)], osem_b
    ).wait()


def _gather(table, idx_flat):
    mesh = plsc.VectorSubcoreMesh(core_axis_name="c", subcore_axis_name="s")
    f = functools.partial(
        pl.kernel,
        mesh=mesh,
        out_type=jax.ShapeDtypeStruct((_B, _DIM), jnp.float32),
        scratch_types=[
            pltpu.VMEM((_I_PER_W,), jnp.int32),
            pltpu.VMEM((_H, _DIM), jnp.float32),
            pltpu.VMEM((_H, _DIM), jnp.float32),
            pltpu.SemaphoreType.DMA,
            pltpu.SemaphoreType.DMA,
            pltpu.SemaphoreType.DMA,
            pltpu.SemaphoreType.DMA,
        ],
    )(_gather_kernel)
    return f(table, idx_flat)


def kernel(t):
    table = _build_table()
    idx = t.T.astype(jnp.int32).reshape(-1)
    out = _gather(table, idx)
    return out.reshape(_T_COLS, _T_ROWS, _DIM).transpose(1, 0, 2)


# speedup vs baseline: 11.2808x; 1.0623x over previous
import functools

import jax
import jax.numpy as jnp
from jax import lax
from jax.experimental import pallas as pl
from jax.experimental.pallas import tpu as pltpu
from jax.experimental.pallas import tpu_sc as plsc

_DIM = 128
_NUM_POS = 100000

_ROW_BLOCK = 2000
_PI = 3.141592653589793
_HALF_PI = _PI / 2


def _fast_sin(x):
    sign = jnp.where(x > _PI, -1.0, 1.0)
    y = jnp.where(x > _PI, x - _PI, x)
    y = jnp.where(y > _HALF_PI, _PI - y, y)
    s = y * y
    p = jnp.float32(1.0 / 362880)
    p = p * s + jnp.float32(-1.0 / 5040)
    p = p * s + jnp.float32(1.0 / 120)
    p = p * s + jnp.float32(-1.0 / 6)
    p = p * s + 1.0
    return sign * y * p


def _table_body(out_ref):
    i = pl.program_id(0)
    r = jax.lax.broadcasted_iota(jnp.int32, (_ROW_BLOCK, _DIM), 0)
    r = (r + i * _ROW_BLOCK).astype(jnp.float32)
    b = r * jnp.float32(1.0 / 10000.0)
    c = jax.lax.broadcasted_iota(jnp.int32, (_ROW_BLOCK, _DIM), 1)
    k = c // 2
    e = k.astype(jnp.float32) * jnp.float32(1.0 / _DIM)
    phase = jnp.where(k == 0, 1.0, jnp.exp2(e * jnp.log2(b)))
    phase = phase + jnp.where(c % 2 == 0, 0.0, _HALF_PI)
    out_ref[...] = _fast_sin(phase)


def _build_table():
    return pl.pallas_call(
        _table_body,
        out_shape=jax.ShapeDtypeStruct((_NUM_POS, _DIM), jnp.float32),
        grid=(_NUM_POS // _ROW_BLOCK,),
        out_specs=pl.BlockSpec((_ROW_BLOCK, _DIM), lambda i: (i, 0)),
    )()


_T_ROWS = 16384
_T_COLS = 50
_B = _T_ROWS * _T_COLS
_NW = 32
_I_PER_W = _T_ROWS // _NW
_H = _I_PER_W // 2


def _gather_kernel(
    table_hbm, idx_hbm, out_hbm, idx_a, idx_b, rows_a, rows_b, isem_a,
    isem_b, gsem_a, gsem_b, osem_a, osem_b
):
    nc = 2
    wid = lax.axis_index("s") * nc + lax.axis_index("c")
    base_i = wid * _I_PER_W

    def half(j, idx_v, rows_v, gsem, osem, off):
        @pl.when(j > 0)
        def _():
            pltpu.make_async_copy(
                rows_v,
                out_hbm.at[pl.ds((j - 1) * _T_ROWS + base_i + off, _H)],
                osem,
            ).wait()

        c0 = pltpu.async_copy(
            table_hbm.at[idx_v.at[pl.ds(off, _DIM)]],
            rows_v.at[pl.ds(0, _DIM)],
            gsem,
        )
        c1 = pltpu.async_copy(
            table_hbm.at[idx_v.at[pl.ds(off + _DIM, _DIM)]],
            rows_v.at[pl.ds(_DIM, _DIM)],
            gsem,
        )
        c0.wait()
        c1.wait()
        pltpu.async_copy(
            rows_v, out_hbm.at[pl.ds(j * _T_ROWS + base_i + off, _H)], osem
        )

    pltpu.async_copy(idx_hbm.at[pl.ds(base_i, _I_PER_W)], idx_a, isem_a)
    pltpu.async_copy(
        idx_hbm.at[pl.ds(_T_ROWS + base_i, _I_PER_W)], idx_b, isem_b
    )

    def column_pair(p, carry):
        j0 = 2 * p
        pltpu.make_async_copy(
            idx_hbm.at[pl.ds(j0 * _T_ROWS + base_i, _I_PER_W)], idx_a, isem_a
        ).wait()
        half(j0, idx_a, rows_a, gsem_a, osem_a, 0)
        half(j0, idx_a, rows_b, gsem_b, osem_b, _H)

        @pl.when(p < _T_COLS // 2 - 1)
        def _():
            pltpu.async_copy(
                idx_hbm.at[pl.ds((j0 + 2) * _T_ROWS + base_i, _I_PER_W)],
                idx_a,
                isem_a,
            )

        j1 = j0 + 1
        pltpu.make_async_copy(
            idx_hbm.at[pl.ds(j1 * _T_ROWS + base_i, _I_PER_W)], idx_b, isem_b
        ).wait()
        half(j1, idx_b, rows_a, gsem_a, osem_a, 0)
        half(j1, idx_b, rows_b, gsem_b, osem_b, _H)

        @pl.when(p < _T_COLS // 2 - 1)
        def _():
            pltpu.async_copy(
                idx_hbm.at[pl.ds((j1 + 2) * _T_ROWS + base_i, _I_PER_W)],
                idx_b,
                isem_b,
            )

        return carry

    lax.fori_loop(0, _T_COLS // 2, column_pair, 0)
    j_last = _T_COLS - 1
    pltpu.make_async_copy(
        rows_a, out_hbm.at[pl.ds(j_last * _T_ROWS + base_i, _H)], osem_a
    ).wait()
    pltpu.make_async_copy(
        rows_b, out_hbm.at[pl.ds(j_last * _T_ROWS + base_i + _H, _H)], osem_b
    ).wait()


def _gather(table, idx_flat):
    mesh = plsc.VectorSubcoreMesh(core_axis_name="c", subcore_axis_name="s")
    f = functools.partial(
        pl.kernel,
        mesh=mesh,
        out_type=jax.ShapeDtypeStruct((_B, _DIM), jnp.float32),
        scratch_types=[
            pltpu.VMEM((_I_PER_W,), jnp.int32),
            pltpu.VMEM((_I_PER_W,), jnp.int32),
            pltpu.VMEM((_H, _DIM), jnp.float32),
            pltpu.VMEM((_H, _DIM), jnp.float32),
            pltpu.SemaphoreType.DMA,
            pltpu.SemaphoreType.DMA,
            pltpu.SemaphoreType.DMA,
            pltpu.SemaphoreType.DMA,
            pltpu.SemaphoreType.DMA,
            pltpu.SemaphoreType.DMA,
        ],
    )(_gather_kernel)
    return f(table, idx_flat)


def kernel(t):
    table = _build_table()
    idx = t.T.astype(jnp.int32).reshape(-1)
    out = _gather(table, idx)
    return out.reshape(_T_COLS, _T_ROWS, _DIM).transpose(1, 0, 2)


# speedup vs baseline: 11.3596x; 1.0070x over previous
import functools

import jax
import jax.numpy as jnp
from jax import lax
from jax.experimental import pallas as pl
from jax.experimental.pallas import tpu as pltpu
from jax.experimental.pallas import tpu_sc as plsc

_DIM = 128
_NUM_POS = 100000

_ROW_BLOCK = 2000
_PI = 3.141592653589793
_HALF_PI = _PI / 2


def _fast_sin(x):
    sign = jnp.where(x > _PI, -1.0, 1.0)
    y = jnp.where(x > _PI, x - _PI, x)
    y = jnp.where(y > _HALF_PI, _PI - y, y)
    s = y * y
    p = jnp.float32(1.0 / 362880)
    p = p * s + jnp.float32(-1.0 / 5040)
    p = p * s + jnp.float32(1.0 / 120)
    p = p * s + jnp.float32(-1.0 / 6)
    p = p * s + 1.0
    return sign * y * p


def _table_body(out_ref):
    i = pl.program_id(0)
    r = jax.lax.broadcasted_iota(jnp.int32, (_ROW_BLOCK, _DIM), 0)
    r = (r + i * _ROW_BLOCK).astype(jnp.float32)
    b = r * jnp.float32(1.0 / 10000.0)
    c = jax.lax.broadcasted_iota(jnp.int32, (_ROW_BLOCK, _DIM), 1)
    k = c // 2
    e = k.astype(jnp.float32) * jnp.float32(1.0 / _DIM)
    phase = jnp.where(k == 0, 1.0, jnp.exp2(e * jnp.log2(b)))
    phase = phase + jnp.where(c % 2 == 0, 0.0, _HALF_PI)
    out_ref[...] = _fast_sin(phase)


def _build_table():
    return pl.pallas_call(
        _table_body,
        out_shape=jax.ShapeDtypeStruct((_NUM_POS, _DIM), jnp.float32),
        grid=(_NUM_POS // _ROW_BLOCK,),
        out_specs=pl.BlockSpec((_ROW_BLOCK, _DIM), lambda i: (i, 0)),
    )()


_T_ROWS = 16384
_T_COLS = 50
_B = _T_ROWS * _T_COLS
_NW = 32
_I_PER_W = _T_ROWS // _NW
_H = _I_PER_W // 2


_NBUF = 4


def _gather_kernel(
    table_hbm, idx_hbm, out_hbm, idx_a, idx_b,
    rows_0, rows_1, rows_2, rows_3, isem_a, isem_b,
    gsem_0, gsem_1, gsem_2, gsem_3, osem_0, osem_1, osem_2, osem_3
):
    nc = 2
    wid = lax.axis_index("s") * nc + lax.axis_index("c")
    base_i = wid * _I_PER_W
    rows = (rows_0, rows_1, rows_2, rows_3)
    gsems = (gsem_0, gsem_1, gsem_2, gsem_3)
    osems = (osem_0, osem_1, osem_2, osem_3)

    def do_column(j, idx_v):
        handles = []
        for u in range(_NBUF):
            @pl.when(j > 0)
            def _(u=u):
                pltpu.make_async_copy(
                    rows[u],
                    out_hbm.at[
                        pl.ds((j - 1) * _T_ROWS + base_i + u * _DIM, _DIM)
                    ],
                    osems[u],
                ).wait()

            handles.append(
                pltpu.async_copy(
                    table_hbm.at[idx_v.at[pl.ds(u * _DIM, _DIM)]],
                    rows[u],
                    gsems[u],
                )
            )
        for u in range(_NBUF):
            handles[u].wait()
            pltpu.async_copy(
                rows[u],
                out_hbm.at[pl.ds(j * _T_ROWS + base_i + u * _DIM, _DIM)],
                osems[u],
            )

    pltpu.async_copy(idx_hbm.at[pl.ds(base_i, _I_PER_W)], idx_a, isem_a)
    pltpu.async_copy(
        idx_hbm.at[pl.ds(_T_ROWS + base_i, _I_PER_W)], idx_b, isem_b
    )

    def column_pair(p, carry):
        for j, idx_v, isem in ((2 * p, idx_a, isem_a), (2 * p + 1, idx_b, isem_b)):
            pltpu.make_async_copy(
                idx_hbm.at[pl.ds(j * _T_ROWS + base_i, _I_PER_W)], idx_v, isem
            ).wait()
            do_column(j, idx_v)

            @pl.when(p < _T_COLS // 2 - 1)
            def _(j=j, idx_v=idx_v, isem=isem):
                pltpu.async_copy(
                    idx_hbm.at[pl.ds((j + 2) * _T_ROWS + base_i, _I_PER_W)],
                    idx_v,
                    isem,
                )

        return carry

    lax.fori_loop(0, _T_COLS // 2, column_pair, 0)
    j_last = _T_COLS - 1
    for u in range(_NBUF):
        pltpu.make_async_copy(
            rows[u],
            out_hbm.at[pl.ds(j_last * _T_ROWS + base_i + u * _DIM, _DIM)],
            osems[u],
        ).wait()


def _gather(table, idx_flat):
    mesh = plsc.VectorSubcoreMesh(core_axis_name="c", subcore_axis_name="s")
    f = functools.partial(
        pl.kernel,
        mesh=mesh,
        out_type=jax.ShapeDtypeStruct((_B, _DIM), jnp.float32),
        scratch_types=(
            [pltpu.VMEM((_I_PER_W,), jnp.int32)] * 2
            + [pltpu.VMEM((_DIM, _DIM), jnp.float32)] * _NBUF
            + [pltpu.SemaphoreType.DMA] * (2 + 2 * _NBUF)
        ),
    )(_gather_kernel)
    return f(table, idx_flat)


def kernel(t):
    table = _build_table()
    idx = t.T.astype(jnp.int32).reshape(-1)
    out = _gather(table, idx)
    return out.reshape(_T_COLS, _T_ROWS, _DIM).transpose(1, 0, 2)


# speedup vs baseline: 11.4305x; 1.0062x over previous
import functools

import jax
import jax.numpy as jnp
from jax import lax
from jax.experimental import pallas as pl
from jax.experimental.pallas import tpu as pltpu
from jax.experimental.pallas import tpu_sc as plsc

_DIM = 128
_NUM_POS = 100000

_ROW_BLOCK = 4000
_PI = 3.141592653589793
_HALF_PI = _PI / 2


def _fast_sin(x):
    sign = jnp.where(x > _PI, -1.0, 1.0)
    y = jnp.where(x > _PI, x - _PI, x)
    y = jnp.where(y > _HALF_PI, _PI - y, y)
    s = y * y
    p = jnp.float32(1.0 / 362880)
    p = p * s + jnp.float32(-1.0 / 5040)
    p = p * s + jnp.float32(1.0 / 120)
    p = p * s + jnp.float32(-1.0 / 6)
    p = p * s + 1.0
    return sign * y * p


def _table_body(out_ref):
    i = pl.program_id(0)
    r = jax.lax.broadcasted_iota(jnp.int32, (_ROW_BLOCK, _DIM), 0)
    r = (r + i * _ROW_BLOCK).astype(jnp.float32)
    b = r * jnp.float32(1.0 / 10000.0)
    c = jax.lax.broadcasted_iota(jnp.int32, (_ROW_BLOCK, _DIM), 1)
    k = c // 2
    e = k.astype(jnp.float32) * jnp.float32(1.0 / _DIM)
    phase = jnp.where(k == 0, 1.0, jnp.exp2(e * jnp.log2(b)))
    phase = phase + jnp.where(c % 2 == 0, 0.0, _HALF_PI)
    out_ref[...] = _fast_sin(phase)


def _build_table():
    return pl.pallas_call(
        _table_body,
        out_shape=jax.ShapeDtypeStruct((_NUM_POS, _DIM), jnp.float32),
        grid=(_NUM_POS // _ROW_BLOCK,),
        out_specs=pl.BlockSpec((_ROW_BLOCK, _DIM), lambda i: (i, 0)),
    )()


_T_ROWS = 16384
_T_COLS = 50
_B = _T_ROWS * _T_COLS
_NW = 32
_I_PER_W = _T_ROWS // _NW
_H = _I_PER_W // 2


_NBUF = 4


def _gather_kernel(
    table_hbm, idx_hbm, out_hbm, idx_a, idx_b,
    rows_0, rows_1, rows_2, rows_3, isem_a, isem_b,
    gsem_0, gsem_1, gsem_2, gsem_3, osem_0, osem_1, osem_2, osem_3
):
    nc = 2
    wid = lax.axis_index("s") * nc + lax.axis_index("c")
    base_i = wid * _I_PER_W
    rows = (rows_0, rows_1, rows_2, rows_3)
    gsems = (gsem_0, gsem_1, gsem_2, gsem_3)
    osems = (osem_0, osem_1, osem_2, osem_3)

    def do_column(j, idx_v):
        handles = []
        for u in range(_NBUF):
            @pl.when(j > 0)
            def _(u=u):
                pltpu.make_async_copy(
                    rows[u],
                    out_hbm.at[
                        pl.ds((j - 1) * _T_ROWS + base_i + u * _DIM, _DIM)
                    ],
                    osems[u],
                ).wait()

            handles.append(
                pltpu.async_copy(
                    table_hbm.at[idx_v.at[pl.ds(u * _DIM, _DIM)]],
                    rows[u],
                    gsems[u],
                )
            )
        for u in range(_NBUF):
            handles[u].wait()
            pltpu.async_copy(
                rows[u],
                out_hbm.at[pl.ds(j * _T_ROWS + base_i + u * _DIM, _DIM)],
                osems[u],
            )

    pltpu.async_copy(idx_hbm.at[pl.ds(base_i, _I_PER_W)], idx_a, isem_a)
    pltpu.async_copy(
        idx_hbm.at[pl.ds(_T_ROWS + base_i, _I_PER_W)], idx_b, isem_b
    )

    def column_pair(p, carry):
        for j, idx_v, isem in ((2 * p, idx_a, isem_a), (2 * p + 1, idx_b, isem_b)):
            pltpu.make_async_copy(
                idx_hbm.at[pl.ds(j * _T_ROWS + base_i, _I_PER_W)], idx_v, isem
            ).wait()
            do_column(j, idx_v)

            @pl.when(p < _T_COLS // 2 - 1)
            def _(j=j, idx_v=idx_v, isem=isem):
                pltpu.async_copy(
                    idx_hbm.at[pl.ds((j + 2) * _T_ROWS + base_i, _I_PER_W)],
                    idx_v,
                    isem,
                )

        return carry

    lax.fori_loop(0, _T_COLS // 2, column_pair, 0)
    j_last = _T_COLS - 1
    for u in range(_NBUF):
        pltpu.make_async_copy(
            rows[u],
            out_hbm.at[pl.ds(j_last * _T_ROWS + base_i + u * _DIM, _DIM)],
            osems[u],
        ).wait()


def _gather(table, idx_flat):
    mesh = plsc.VectorSubcoreMesh(core_axis_name="c", subcore_axis_name="s")
    f = functools.partial(
        pl.kernel,
        mesh=mesh,
        out_type=jax.ShapeDtypeStruct((_B, _DIM), jnp.float32),
        scratch_types=(
            [pltpu.VMEM((_I_PER_W,), jnp.int32)] * 2
            + [pltpu.VMEM((_DIM, _DIM), jnp.float32)] * _NBUF
            + [pltpu.SemaphoreType.DMA] * (2 + 2 * _NBUF)
        ),
    )(_gather_kernel)
    return f(table, idx_flat)


def kernel(t):
    table = _build_table()
    idx = t.T.astype(jnp.int32).reshape(-1)
    out = _gather(table, idx)
    return out.reshape(_T_COLS, _T_ROWS, _DIM).transpose(1, 0, 2)
